# Initial kernel scaffold; baseline (speedup 1.0000x reference)
import jax
import jax.numpy as jnp
from jax.experimental import pallas as pl


def kernel(x, edge_index, W1, b1, W2, b2):
    raise NotImplementedError("write your pallas kernel here")


# speedup vs baseline: 14.8424x; 14.8424x over previous
import functools

import jax
import jax.numpy as jnp
from jax import lax
from jax.experimental import pallas as pl
from jax.experimental.pallas import tpu as pltpu
from jax.experimental.pallas import tpu_sc as plsc

N = 10000
NP = 10240
E = 320000
D = 128
NC = 2
NS = 16
NW = NC * NS
EPT = E // NW
CH = 80
NCHUNK = EPT // CH
RPT = NP // NS
_f32 = jnp.float32
_i32 = jnp.int32

_MESH = plsc.VectorSubcoreMesh(
    core_axis_name="c", subcore_axis_name="s", num_cores=NC, num_subcores=NS
)


@functools.partial(
    pl.kernel,
    out_type=jax.ShapeDtypeStruct((NC, NP, D), _f32),
    mesh=_MESH,
    scratch_types=[
        pltpu.VMEM_SHARED((NP, D), _f32),
        pltpu.VMEM((NCHUNK, CH), _i32),
        pltpu.VMEM((NCHUNK, CH), _i32),
        pltpu.VMEM((CH, D), _f32),
    ],
)
def _prop_sc(t_hbm, row_hbm, col_hbm, out_hbm, acc, ridx, cidx, gbuf):
    c = lax.axis_index("c")
    s = lax.axis_index("s")
    wid = c * NS + s

    def _zrow(i, carry):
        for j in range(D // 16):
            gbuf[i, pl.ds(j * 16, 16)] = jnp.zeros((16,), _f32)
        return carry

    lax.fori_loop(0, CH, _zrow, 0)
    for r in range(RPT // CH):
        pltpu.sync_copy(gbuf, acc.at[pl.ds(s * RPT + r * CH, CH)])
    pltpu.sync_copy(row_hbm.at[wid], ridx)
    pltpu.sync_copy(col_hbm.at[wid], cidx)
    plsc.subcore_barrier()

    def _edge_chunk(k, carry):
        pltpu.sync_copy(t_hbm.at[ridx.at[k]], gbuf)
        pltpu.sync_copy(gbuf, acc.at[cidx.at[k]], add=True)
        return carry

    lax.fori_loop(0, NCHUNK, _edge_chunk, 0)
    plsc.subcore_barrier()
    pltpu.sync_copy(acc.at[pl.ds(s * RPT, RPT)], out_hbm.at[c, pl.ds(s * RPT, RPT)])


@functools.partial(
    pl.kernel,
    out_type=jax.ShapeDtypeStruct((NC, NP, D), _f32),
    mesh=_MESH,
    scratch_types=[
        pltpu.VMEM_SHARED((NP, D), _f32),
        pltpu.VMEM((NCHUNK, CH), _i32),
        pltpu.VMEM((CH, D), _f32),
    ],
)
def _deg_sc(col_hbm, out_hbm, acc, cidx, gbuf):
    c = lax.axis_index("c")
    s = lax.axis_index("s")
    wid = c * NS + s

    def _fill(val):
        def _row(i, carry):
            for j in range(D // 16):
                gbuf[i, pl.ds(j * 16, 16)] = jnp.full((16,), val, _f32)
            return carry
        lax.fori_loop(0, CH, _row, 0)

    _fill(0.0)
    for r in range(RPT // CH):
        pltpu.sync_copy(gbuf, acc.at[pl.ds(s * RPT + r * CH, CH)])
    _fill(1.0)
    pltpu.sync_copy(col_hbm.at[wid], cidx)
    plsc.subcore_barrier()

    def _edge_chunk(k, carry):
        pltpu.sync_copy(gbuf, acc.at[cidx.at[k]], add=True)
        return carry

    lax.fori_loop(0, NCHUNK, _edge_chunk, 0)
    plsc.subcore_barrier()
    pltpu.sync_copy(acc.at[pl.ds(s * RPT, RPT)], out_hbm.at[c, pl.ds(s * RPT, RPT)])


def _pre_body(p_ref, x_ref, t0_ref, dis_ref, dinv_ref):
    deg = 1.0 + p_ref[0, 0:N, 0:1] + p_ref[1, 0:N, 0:1]
    dis = lax.rsqrt(deg)
    dis_ref[...] = dis
    dinv_ref[...] = 1.0 / deg
    t0_ref[...] = x_ref[...] * dis


_pre_tc = pl.pallas_call(
    _pre_body,
    out_shape=(
        jax.ShapeDtypeStruct((N, D), _f32),
        jax.ShapeDtypeStruct((N, 1), _f32),
        jax.ShapeDtypeStruct((N, 1), _f32),
    ),
)


def _mid_body(p_ref, t_ref, dinv_ref, o_ref):
    o_ref[...] = dinv_ref[...] * (p_ref[0, 0:N] + p_ref[1, 0:N] + t_ref[...])


_mid_tc = pl.pallas_call(_mid_body, out_shape=jax.ShapeDtypeStruct((N, D), _f32))


def _mm_body(relu_scale, p_ref, t_ref, dis_ref, w_ref, b_ref, o_ref):
    u = dis_ref[...] * (p_ref[0, 0:N] + p_ref[1, 0:N] + t_ref[...])
    y = jnp.dot(u, w_ref[...], preferred_element_type=_f32) + b_ref[...]
    if relu_scale:
        o_ref[...] = dis_ref[...] * jnp.maximum(y, 0.0)
    else:
        o_ref[...] = y


_mm_relu_tc = pl.pallas_call(
    functools.partial(_mm_body, True), out_shape=jax.ShapeDtypeStruct((N, D), _f32)
)
_mm_plain_tc = pl.pallas_call(
    functools.partial(_mm_body, False), out_shape=jax.ShapeDtypeStruct((N, D), _f32)
)


def kernel(x, edge_index, W1, b1, W2, b2):
    row3 = edge_index[0].astype(_i32).reshape(NW, NCHUNK, CH)
    col3 = edge_index[1].astype(_i32).reshape(NW, NCHUNK, CH)
    degp = _deg_sc(col3)
    t0, dis, dinv = _pre_tc(degp, x)
    a = _prop_sc(t0, row3, col3)
    t1 = _mid_tc(a, t0, dinv)
    a = _prop_sc(t1, row3, col3)
    v0 = _mm_relu_tc(a, t1, dis, W1, b1.reshape(1, D))
    a = _prop_sc(v0, row3, col3)
    v1 = _mid_tc(a, v0, dinv)
    a = _prop_sc(v1, row3, col3)
    return _mm_plain_tc(a, v1, dis, W2, b2.reshape(1, D))


# speedup vs baseline: 18.5965x; 1.2529x over previous
import functools

import jax
import jax.numpy as jnp
from jax import lax
from jax.experimental import pallas as pl
from jax.experimental.pallas import tpu as pltpu
from jax.experimental.pallas import tpu_sc as plsc

N = 10000
NP = 10240
E = 320000
D = 128
NC = 2
NS = 16
NW = NC * NS
EPT = E // NW
CH = 80
NCHUNK = EPT // CH
RPT = NP // NS
_f32 = jnp.float32
_i32 = jnp.int32

_MESH = plsc.VectorSubcoreMesh(
    core_axis_name="c", subcore_axis_name="s", num_cores=NC, num_subcores=NS
)


@functools.partial(
    pl.kernel,
    out_type=jax.ShapeDtypeStruct((NC, NP, D), _f32),
    mesh=_MESH,
    scratch_types=[
        pltpu.VMEM_SHARED((NP, D), _f32),
        pltpu.VMEM((NCHUNK, CH), _i32),
        pltpu.VMEM((2, CH), _i32),
        pltpu.VMEM((2, CH), _i32),
        pltpu.VMEM((CH, D), _f32),
        pltpu.VMEM((CH, D), _f32),
        pltpu.SemaphoreType.DMA,
    ],
)
def _prop_sc(t_hbm, pidx_hbm, out_hbm, acc, pidx, st0, st1, gb0, gb1, sm0):
    c = lax.axis_index("c")
    s = lax.axis_index("s")
    wid = c * NS + s

    def _zrow(i, carry):
        for j in range(D // 16):
            gb0[i, pl.ds(j * 16, 16)] = jnp.zeros((16,), _f32)
        return carry

    lax.fori_loop(0, CH, _zrow, 0)
    for r in range(RPT // CH):
        pltpu.sync_copy(gb0, acc.at[pl.ds(s * RPT + r * CH, CH)])
    pltpu.sync_copy(pidx_hbm.at[wid], pidx)
    plsc.subcore_barrier()

    def _unpack(k, st):
        for j in range(CH // 16):
            w = pidx[k, pl.ds(j * 16, 16)]
            st[0, pl.ds(j * 16, 16)] = w & jnp.int32(0xFFFF)
            st[1, pl.ds(j * 16, 16)] = lax.shift_right_logical(w, 16)

    def _issue(st, buf):
        pltpu.async_copy(t_hbm.at[st.at[0]], buf, sm0)

    def _drain(st, buf):
        pltpu.make_async_copy(t_hbm.at[st.at[0]], buf, sm0).wait()

    def _scat(st, buf):
        pltpu.sync_copy(buf, acc.at[st.at[1]], add=True)

    _unpack(0, st0)
    _issue(st0, gb0)

    def _pair(i, carry):
        _drain(st0, gb0)
        _unpack(2 * i + 1, st1)
        _issue(st1, gb1)
        _scat(st0, gb0)
        _drain(st1, gb1)
        _unpack(2 * i + 2, st0)
        _issue(st0, gb0)
        _scat(st1, gb1)
        return carry

    lax.fori_loop(0, (NCHUNK - 1) // 2, _pair, 0)
    _drain(st0, gb0)
    _scat(st0, gb0)
    plsc.subcore_barrier()
    pltpu.sync_copy(acc.at[pl.ds(s * RPT, RPT)], out_hbm.at[c, pl.ds(s * RPT, RPT)])


@functools.partial(
    pl.kernel,
    out_type=jax.ShapeDtypeStruct((NC, NP, D), _f32),
    mesh=_MESH,
    scratch_types=[
        pltpu.VMEM_SHARED((NP, D), _f32),
        pltpu.VMEM((NCHUNK, CH), _i32),
        pltpu.VMEM((CH, D), _f32),
    ],
)
def _deg_sc(col_hbm, out_hbm, acc, cidx, gbuf):
    c = lax.axis_index("c")
    s = lax.axis_index("s")
    wid = c * NS + s

    def _fill(val):
        def _row(i, carry):
            for j in range(D // 16):
                gbuf[i, pl.ds(j * 16, 16)] = jnp.full((16,), val, _f32)
            return carry
        lax.fori_loop(0, CH, _row, 0)

    _fill(0.0)
    for r in range(RPT // CH):
        pltpu.sync_copy(gbuf, acc.at[pl.ds(s * RPT + r * CH, CH)])
    _fill(1.0)
    pltpu.sync_copy(col_hbm.at[wid], cidx)
    plsc.subcore_barrier()

    def _edge_chunk(k, carry):
        pltpu.sync_copy(gbuf, acc.at[cidx.at[k]], add=True)
        return carry

    lax.fori_loop(0, NCHUNK, _edge_chunk, 0)
    plsc.subcore_barrier()
    pltpu.sync_copy(acc.at[pl.ds(s * RPT, RPT)], out_hbm.at[c, pl.ds(s * RPT, RPT)])


def _pre_body(p_ref, x_ref, t0_ref, dis_ref, dinv_ref):
    deg = 1.0 + p_ref[0, 0:N, 0:1] + p_ref[1, 0:N, 0:1]
    dis = lax.rsqrt(deg)
    dis_ref[...] = dis
    dinv_ref[...] = 1.0 / deg
    t0_ref[...] = x_ref[...] * dis


_pre_tc = pl.pallas_call(
    _pre_body,
    out_shape=(
        jax.ShapeDtypeStruct((N, D), _f32),
        jax.ShapeDtypeStruct((N, 1), _f32),
        jax.ShapeDtypeStruct((N, 1), _f32),
    ),
)


def _mid_body(p_ref, t_ref, dinv_ref, o_ref):
    o_ref[...] = dinv_ref[...] * (p_ref[0, 0:N] + p_ref[1, 0:N] + t_ref[...])


_mid_tc = pl.pallas_call(_mid_body, out_shape=jax.ShapeDtypeStruct((N, D), _f32))


def _mm_body(relu_scale, p_ref, t_ref, dis_ref, w_ref, b_ref, o_ref):
    u = dis_ref[...] * (p_ref[0, 0:N] + p_ref[1, 0:N] + t_ref[...])
    y = jnp.dot(u, w_ref[...], preferred_element_type=_f32) + b_ref[...]
    if relu_scale:
        o_ref[...] = dis_ref[...] * jnp.maximum(y, 0.0)
    else:
        o_ref[...] = y


_mm_relu_tc = pl.pallas_call(
    functools.partial(_mm_body, True), out_shape=jax.ShapeDtypeStruct((N, D), _f32)
)
_mm_plain_tc = pl.pallas_call(
    functools.partial(_mm_body, False), out_shape=jax.ShapeDtypeStruct((N, D), _f32)
)


def kernel(x, edge_index, W1, b1, W2, b2):
    row = edge_index[0].astype(_i32)
    col = edge_index[1].astype(_i32)
    col3 = col.reshape(NW, NCHUNK, CH)
    pidx3 = (row | (col << 16)).reshape(NW, NCHUNK, CH)
    degp = _deg_sc(col3)
    t0, dis, dinv = _pre_tc(degp, x)
    a = _prop_sc(t0, pidx3)
    t1 = _mid_tc(a, t0, dinv)
    a = _prop_sc(t1, pidx3)
    v0 = _mm_relu_tc(a, t1, dis, W1, b1.reshape(1, D))
    a = _prop_sc(v0, pidx3)
    v1 = _mid_tc(a, v0, dinv)
    a = _prop_sc(v1, pidx3)
    return _mm_plain_tc(a, v1, dis, W2, b2.reshape(1, D))
